# baseline (speedup 1.0000x reference)
import functools

import jax
import jax.numpy as jnp
from jax.experimental import pallas as pl
from jax.experimental.pallas import tpu as pltpu

_BM = 128
_NOPS = 4


def _body(a0, a1, a2, a3, e_ref, o_ref):
    e_bf = e_ref[...].astype(jnp.bfloat16)
    i = pl.program_id(0)
    for j, aref in enumerate((a0, a1, a2, a3)):
        @pl.when(i % _NOPS == j)
        def _(aref=aref):
            o_ref[...] = jax.lax.dot_general(
                aref[...], e_bf,
                dimension_numbers=(((1,), (0,)), ((), ())),
                preferred_element_type=jnp.float32,
            )


def _a_index(i, j, nblocks):
    period = jnp.minimum((i + (_NOPS - 1) - j) // _NOPS, nblocks // _NOPS - 1)
    return (j + _NOPS * period, 0)


def kernel(matrix_parents, Epsilon):
    M, K = matrix_parents.shape
    _, N = Epsilon.shape
    nblocks = M // _BM
    a_specs = [
        pl.BlockSpec((_BM, K), functools.partial(_a_index, j=j, nblocks=nblocks))
        for j in range(_NOPS)
    ]
    return pl.pallas_call(
        _body,
        grid=(nblocks,),
        in_specs=a_specs + [pl.BlockSpec((K, N), lambda i: (0, 0))],
        out_specs=pl.BlockSpec((_BM, N), lambda i: (i, 0)),
        out_shape=jax.ShapeDtypeStruct((M, N), jnp.float32),
        compiler_params=pltpu.CompilerParams(
            dimension_semantics=("arbitrary",),
            disable_bounds_checks=True,
        ),
    )(matrix_parents, matrix_parents, matrix_parents, matrix_parents, Epsilon)


# speedup vs baseline: 1.2927x; 1.2927x over previous
import jax
import jax.numpy as jnp
from jax.experimental import pallas as pl
from jax.experimental.pallas import tpu as pltpu

_BM = 256
_SUB = 4


def _body(a_ref, e_ref, o_ref):
    e_bf = e_ref[...].astype(jnp.bfloat16)
    h = _BM // _SUB
    for s in range(_SUB):
        o_ref[pl.ds(s * h, h)] = jax.lax.dot_general(
            a_ref[pl.ds(s * h, h)], e_bf,
            dimension_numbers=(((1,), (0,)), ((), ())),
            preferred_element_type=jnp.float32,
        )


def kernel(matrix_parents, Epsilon):
    M, K = matrix_parents.shape
    _, N = Epsilon.shape
    return pl.pallas_call(
        _body,
        grid=(M // _BM,),
        in_specs=[
            pl.BlockSpec((_BM, K), lambda i: (i, 0)),
            pl.BlockSpec((K, N), lambda i: (0, 0)),
        ],
        out_specs=pl.BlockSpec((_BM, N), lambda i: (i, 0)),
        out_shape=jax.ShapeDtypeStruct((M, N), jnp.float32),
        compiler_params=pltpu.CompilerParams(
            dimension_semantics=("arbitrary",),
            disable_bounds_checks=True,
            skip_device_barrier=True,
        ),
    )(matrix_parents, Epsilon)
